# Initial kernel scaffold; baseline (speedup 1.0000x reference)
import jax
import jax.numpy as jnp
from jax.experimental import pallas as pl


def kernel(brawler_ids, table):
    raise NotImplementedError("write your pallas kernel here")


# speedup vs baseline: 1.1094x; 1.1094x over previous
import functools

import jax
import jax.numpy as jnp
from jax import lax
from jax.experimental import pallas as pl
from jax.experimental.pallas import tpu as pltpu
from jax.experimental.pallas import tpu_sc as plsc

_IDX_PER_STREAM = 128


@functools.lru_cache(maxsize=None)
def _make_gather(n_rows: int, vocab: int, dim: int):
  info = plsc.get_sparse_core_info()
  nc, ns = info.num_cores, info.num_subcores
  nw = nc * ns
  assert n_rows % (nw * _IDX_PER_STREAM) == 0
  rows_per_w = n_rows // nw
  k_streams = 20
  chunk = k_streams * _IDX_PER_STREAM
  assert rows_per_w % chunk == 0
  n_chunks = rows_per_w // chunk
  mesh = plsc.VectorSubcoreMesh(core_axis_name="c", subcore_axis_name="s")

  @functools.partial(
      pl.kernel,
      mesh=mesh,
      compiler_params=pltpu.CompilerParams(use_tc_tiling_on_sc=False),
      out_type=jax.ShapeDtypeStruct((n_rows, dim), jnp.float32),
      scratch_types=[
          pltpu.VMEM((chunk,), jnp.int32),
          pltpu.VMEM((chunk, dim), jnp.float32),
          pltpu.SemaphoreType.DMA,
      ],
  )
  def gather(table_hbm, idx_hbm, out_hbm, idx_v, rows_v, sem):
    wid = lax.axis_index("s") * nc + lax.axis_index("c")
    base = wid * rows_per_w

    def body(i, carry):
      off = base + i * chunk
      pltpu.sync_copy(idx_hbm.at[pl.ds(off, chunk)], idx_v)
      copies = [
          pltpu.async_copy(
              table_hbm.at[idx_v.at[pl.ds(j * _IDX_PER_STREAM,
                                          _IDX_PER_STREAM)]],
              rows_v.at[pl.ds(j * _IDX_PER_STREAM, _IDX_PER_STREAM), :],
              sem,
          )
          for j in range(k_streams)
      ]
      for c in copies:
        c.wait()
      pltpu.sync_copy(rows_v, out_hbm.at[pl.ds(off, chunk), :])
      return carry

    lax.fori_loop(0, n_chunks, body, 0)

  return gather


def kernel(brawler_ids, table):
  batch, hist = brawler_ids.shape
  vocab, dim = table.shape
  n_rows = batch * hist
  idx_flat = brawler_ids.astype(jnp.int32).reshape(n_rows)
  out = _make_gather(n_rows, vocab, dim)(table, idx_flat)
  return out.reshape(batch, hist, dim)


# speedup vs baseline: 1.7406x; 1.5689x over previous
import functools

import jax
import jax.numpy as jnp
from jax import lax
from jax.experimental import pallas as pl
from jax.experimental.pallas import tpu as pltpu
from jax.experimental.pallas import tpu_sc as plsc


@functools.lru_cache(maxsize=None)
def _make_gather(batch: int, hist: int, vocab: int, dim: int):
  info = plsc.get_sparse_core_info()
  nc, ns = info.num_cores, info.num_subcores
  nw = nc * ns
  assert batch % nw == 0
  rows_per_w = batch // nw
  chunk = 16
  assert rows_per_w % chunk == 0
  n_chunks = rows_per_w // chunk
  mesh = plsc.VectorSubcoreMesh(core_axis_name="c", subcore_axis_name="s")

  @functools.partial(
      pl.kernel,
      mesh=mesh,
      compiler_params=pltpu.CompilerParams(use_tc_tiling_on_sc=False),
      out_type=jax.ShapeDtypeStruct((batch, hist, dim), jnp.float32),
      scratch_types=[
          pltpu.VMEM((chunk, hist), jnp.int32),
          pltpu.VMEM((chunk, hist, dim), jnp.float32),
          pltpu.SemaphoreType.DMA,
      ],
  )
  def gather(table_hbm, ids_hbm, out_hbm, idx_v, rows_v, sem):
    wid = lax.axis_index("s") * nc + lax.axis_index("c")
    base = wid * rows_per_w

    def body(i, carry):
      r0 = base + i * chunk
      pltpu.sync_copy(ids_hbm.at[pl.ds(r0, chunk), :], idx_v)
      copies = [
          pltpu.async_copy(table_hbm.at[idx_v.at[r]], rows_v.at[r], sem)
          for r in range(chunk)
      ]
      for c in copies:
        c.wait()
      pltpu.sync_copy(rows_v, out_hbm.at[pl.ds(r0, chunk), :, :])
      return carry

    lax.fori_loop(0, n_chunks, body, 0)

  return gather


def kernel(brawler_ids, table):
  batch, hist = brawler_ids.shape
  vocab, dim = table.shape
  return _make_gather(batch, hist, vocab, dim)(
      table, brawler_ids.astype(jnp.int32))
